# initial kernel scaffold (unmeasured)
import jax
import jax.numpy as jnp
from jax import lax
from jax.experimental import pallas as pl
from jax.experimental.pallas import tpu as pltpu


def kernel(
    x,
):
    def body(*refs):
        pass

    out_shape = jax.ShapeDtypeStruct(..., jnp.float32)
    return pl.pallas_call(body, out_shape=out_shape)(...)


# baseline (device time: 194340 ns/iter reference)
import jax
import jax.numpy as jnp
from jax import lax
from jax.experimental import pallas as pl
from jax.experimental.pallas import tpu as pltpu


def kernel(x):
    m, n = x.shape

    def body(x_ref, out_ref, comm1, comm2, sems):
        my_x = lax.axis_index("x")
        my_y = lax.axis_index("y")
        y_nbr = (my_x, 1 - my_y)
        x_nbr = (1 - my_x, my_y)

        barrier = pltpu.get_barrier_semaphore()
        for nbr in (y_nbr, x_nbr):
            pl.semaphore_signal(
                barrier, inc=1, device_id=nbr,
                device_id_type=pl.DeviceIdType.MESH,
            )
        pl.semaphore_wait(barrier, 2)

        rdma1 = pltpu.make_async_remote_copy(
            src_ref=x_ref,
            dst_ref=comm1,
            send_sem=sems.at[0],
            recv_sem=sems.at[1],
            device_id=y_nbr,
            device_id_type=pl.DeviceIdType.MESH,
        )
        rdma1.start()
        rdma1.wait()
        out_ref[...] = x_ref[...] + comm1[...]

        rdma2 = pltpu.make_async_remote_copy(
            src_ref=out_ref,
            dst_ref=comm2,
            send_sem=sems.at[2],
            recv_sem=sems.at[3],
            device_id=x_nbr,
            device_id_type=pl.DeviceIdType.MESH,
        )
        rdma2.start()
        rdma2.wait()
        out_ref[...] = out_ref[...] + comm2[...]

    return pl.pallas_call(
        body,
        out_shape=jax.ShapeDtypeStruct((m, n), jnp.float32),
        in_specs=[pl.BlockSpec(memory_space=pltpu.VMEM)],
        out_specs=pl.BlockSpec(memory_space=pltpu.VMEM),
        scratch_shapes=[
            pltpu.VMEM((m, n), jnp.float32),
            pltpu.VMEM((m, n), jnp.float32),
            pltpu.SemaphoreType.DMA((4,)),
        ],
        compiler_params=pltpu.CompilerParams(collective_id=0),
    )(x)


# device time: 84179 ns/iter; 2.3087x vs baseline; 2.3087x over previous
import jax
import jax.numpy as jnp
from jax import lax
from jax.experimental import pallas as pl
from jax.experimental.pallas import tpu as pltpu


def kernel(x):
    m, n = x.shape
    h = m // 2
    q = m // 4
    e = m // 8

    def body(x_ref, out_ref, acc_a, acc_b, rv_a1, rv_b1, rv_a2, rv_b2,
             send_sems, recv_sems):
        my_x = lax.axis_index("x")
        my_y = lax.axis_index("y")
        y_nbr = (my_x, 1 - my_y)
        x_nbr = (1 - my_x, my_y)

        def exchange(src, dst, sem_idx, nbr):
            return pltpu.make_async_remote_copy(
                src_ref=src, dst_ref=dst,
                send_sem=send_sems.at[sem_idx],
                recv_sem=recv_sems.at[sem_idx],
                device_id=nbr, device_id_type=pl.DeviceIdType.MESH,
            )

        barrier = pltpu.get_barrier_semaphore()
        for nbr in (y_nbr, x_nbr):
            pl.semaphore_signal(
                barrier, inc=1, device_id=nbr,
                device_id_type=pl.DeviceIdType.MESH,
            )
        pl.semaphore_wait(barrier, 2)

        a1 = exchange(x_ref.at[pl.ds((1 - my_y) * q, q)], rv_a1, 0, y_nbr)
        b1 = exchange(x_ref.at[pl.ds(h + (1 - my_x) * q, q)], rv_b1, 4, x_nbr)
        a1.start()
        b1.start()

        a1.wait_recv()
        acc_a[...] = x_ref[pl.ds(my_y * q, q), :] + rv_a1[...]
        a2 = exchange(acc_a.at[pl.ds((1 - my_x) * e, e)], rv_a2, 1, x_nbr)
        a2.start()

        b1.wait_recv()
        acc_b[...] = x_ref[pl.ds(h + my_x * q, q), :] + rv_b1[...]
        b2 = exchange(acc_b.at[pl.ds((1 - my_y) * e, e)], rv_b2, 5, y_nbr)
        b2.start()

        a_blk = my_y * q + my_x * e
        b_blk = h + my_x * q + my_y * e

        a2.wait_recv()
        out_ref[pl.ds(a_blk, e), :] = acc_a[pl.ds(my_x * e, e), :] + rv_a2[...]
        a3 = exchange(out_ref.at[pl.ds(a_blk, e)],
                      out_ref.at[pl.ds(a_blk, e)], 2, x_nbr)
        a3.start()

        b2.wait_recv()
        out_ref[pl.ds(b_blk, e), :] = acc_b[pl.ds(my_y * e, e), :] + rv_b2[...]
        b3 = exchange(out_ref.at[pl.ds(b_blk, e)],
                      out_ref.at[pl.ds(b_blk, e)], 6, y_nbr)
        b3.start()

        a3.wait_recv()
        a4 = exchange(out_ref.at[pl.ds(my_y * q, q)],
                      out_ref.at[pl.ds(my_y * q, q)], 3, y_nbr)
        a4.start()

        b3.wait_recv()
        b4 = exchange(out_ref.at[pl.ds(h + my_x * q, q)],
                      out_ref.at[pl.ds(h + my_x * q, q)], 7, x_nbr)
        b4.start()

        a4.wait_recv()
        b4.wait_recv()

        for r in (a1, b1, a2, b2, a3, b3, a4, b4):
            r.wait_send()

    return pl.pallas_call(
        body,
        out_shape=jax.ShapeDtypeStruct((m, n), jnp.float32),
        in_specs=[pl.BlockSpec(memory_space=pltpu.VMEM)],
        out_specs=pl.BlockSpec(memory_space=pltpu.VMEM),
        scratch_shapes=[
            pltpu.VMEM((q, n), jnp.float32),
            pltpu.VMEM((q, n), jnp.float32),
            pltpu.VMEM((q, n), jnp.float32),
            pltpu.VMEM((q, n), jnp.float32),
            pltpu.VMEM((e, n), jnp.float32),
            pltpu.VMEM((e, n), jnp.float32),
            pltpu.SemaphoreType.DMA((8,)),
            pltpu.SemaphoreType.DMA((8,)),
        ],
        compiler_params=pltpu.CompilerParams(collective_id=0),
    )(x)


# device time: 82624 ns/iter; 2.3521x vs baseline; 1.0188x over previous
import jax
import jax.numpy as jnp
from jax import lax
from jax.experimental import pallas as pl
from jax.experimental.pallas import tpu as pltpu


def kernel(x):
    m, n = x.shape
    h = m // 2
    q = m // 4
    e = m // 8

    def body(x_ref, out_ref, acc_a, acc_b, rv_a1, rv_b1, rv_a2, rv_b2,
             send_sems, recv_sems):
        my_x = lax.axis_index("x")
        my_y = lax.axis_index("y")
        y_nbr = (my_x, 1 - my_y)
        x_nbr = (1 - my_x, my_y)

        def exchange(src, dst, sem_idx, nbr):
            return pltpu.make_async_remote_copy(
                src_ref=src, dst_ref=dst,
                send_sem=send_sems.at[sem_idx],
                recv_sem=recv_sems.at[sem_idx],
                device_id=nbr, device_id_type=pl.DeviceIdType.MESH,
            )

        barrier = pltpu.get_barrier_semaphore()
        for nbr in (y_nbr, x_nbr):
            pl.semaphore_signal(
                barrier, inc=1, device_id=nbr,
                device_id_type=pl.DeviceIdType.MESH,
            )
        pl.semaphore_wait(barrier, 2)

        sa = (1 - my_y) * q
        oa = my_y * q
        sb = h + (1 - my_x) * q
        ob = h + my_x * q
        a_blk = oa + my_x * e
        b_blk = ob + my_y * e

        a1p1 = exchange(x_ref.at[pl.ds(sa + (1 - my_x) * e, e)],
                        rv_a1.at[pl.ds((1 - my_x) * e, e)], 0, y_nbr)
        b1p1 = exchange(x_ref.at[pl.ds(sb + (1 - my_y) * e, e)],
                        rv_b1.at[pl.ds((1 - my_y) * e, e)], 1, x_nbr)
        a1p2 = exchange(x_ref.at[pl.ds(sa + my_x * e, e)],
                        rv_a1.at[pl.ds(my_x * e, e)], 2, y_nbr)
        b1p2 = exchange(x_ref.at[pl.ds(sb + my_y * e, e)],
                        rv_b1.at[pl.ds(my_y * e, e)], 3, x_nbr)
        a1p1.start()
        b1p1.start()
        a1p2.start()
        b1p2.start()

        a1p1.wait_recv()
        acc_a[pl.ds((1 - my_x) * e, e), :] = (
            x_ref[pl.ds(oa + (1 - my_x) * e, e), :]
            + rv_a1[pl.ds((1 - my_x) * e, e), :])
        a2 = exchange(acc_a.at[pl.ds((1 - my_x) * e, e)], rv_a2, 4, x_nbr)
        a2.start()

        b1p1.wait_recv()
        acc_b[pl.ds((1 - my_y) * e, e), :] = (
            x_ref[pl.ds(ob + (1 - my_y) * e, e), :]
            + rv_b1[pl.ds((1 - my_y) * e, e), :])
        b2 = exchange(acc_b.at[pl.ds((1 - my_y) * e, e)], rv_b2, 5, y_nbr)
        b2.start()

        a1p2.wait_recv()
        acc_a[pl.ds(my_x * e, e), :] = (
            x_ref[pl.ds(oa + my_x * e, e), :] + rv_a1[pl.ds(my_x * e, e), :])
        b1p2.wait_recv()
        acc_b[pl.ds(my_y * e, e), :] = (
            x_ref[pl.ds(ob + my_y * e, e), :] + rv_b1[pl.ds(my_y * e, e), :])

        a2.wait_recv()
        out_ref[pl.ds(a_blk, e), :] = acc_a[pl.ds(my_x * e, e), :] + rv_a2[...]
        a4p1 = exchange(out_ref.at[pl.ds(a_blk, e)],
                        out_ref.at[pl.ds(a_blk, e)], 6, y_nbr)
        a3 = exchange(out_ref.at[pl.ds(a_blk, e)],
                      out_ref.at[pl.ds(a_blk, e)], 7, x_nbr)
        a4p1.start()
        a3.start()

        b2.wait_recv()
        out_ref[pl.ds(b_blk, e), :] = acc_b[pl.ds(my_y * e, e), :] + rv_b2[...]
        b4p1 = exchange(out_ref.at[pl.ds(b_blk, e)],
                        out_ref.at[pl.ds(b_blk, e)], 8, x_nbr)
        b3 = exchange(out_ref.at[pl.ds(b_blk, e)],
                      out_ref.at[pl.ds(b_blk, e)], 9, y_nbr)
        b4p1.start()
        b3.start()

        a3.wait_recv()
        a4p2 = exchange(out_ref.at[pl.ds(oa + (1 - my_x) * e, e)],
                        out_ref.at[pl.ds(oa + (1 - my_x) * e, e)], 10, y_nbr)
        a4p2.start()

        b3.wait_recv()
        b4p2 = exchange(out_ref.at[pl.ds(ob + (1 - my_y) * e, e)],
                        out_ref.at[pl.ds(ob + (1 - my_y) * e, e)], 11, x_nbr)
        b4p2.start()

        a4p1.wait_recv()
        a4p2.wait_recv()
        b4p1.wait_recv()
        b4p2.wait_recv()

        for r in (a1p1, b1p1, a1p2, b1p2, a2, b2,
                  a4p1, a3, b4p1, b3, a4p2, b4p2):
            r.wait_send()

    return pl.pallas_call(
        body,
        out_shape=jax.ShapeDtypeStruct((m, n), jnp.float32),
        in_specs=[pl.BlockSpec(memory_space=pltpu.VMEM)],
        out_specs=pl.BlockSpec(memory_space=pltpu.VMEM),
        scratch_shapes=[
            pltpu.VMEM((q, n), jnp.float32),
            pltpu.VMEM((q, n), jnp.float32),
            pltpu.VMEM((q, n), jnp.float32),
            pltpu.VMEM((q, n), jnp.float32),
            pltpu.VMEM((e, n), jnp.float32),
            pltpu.VMEM((e, n), jnp.float32),
            pltpu.SemaphoreType.DMA((12,)),
            pltpu.SemaphoreType.DMA((12,)),
        ],
        compiler_params=pltpu.CompilerParams(collective_id=0),
    )(x)


# device time: 79422 ns/iter; 2.4469x vs baseline; 1.0403x over previous
import jax
import jax.numpy as jnp
from jax import lax
from jax.experimental import pallas as pl
from jax.experimental.pallas import tpu as pltpu

NC = 2


def kernel(x):
    m, n = x.shape
    h = m // 2
    q = m // 4
    e = m // 8
    cw = n // NC

    def body(x_ref, out_ref, acc_a, acc_b, rv_a1, rv_b1, rv_a2, rv_b2,
             send_sems, recv_sems):
        my_x = lax.axis_index("x")
        my_y = lax.axis_index("y")
        y_nbr = (my_x, 1 - my_y)
        x_nbr = (1 - my_x, my_y)

        sem_ctr = [0]

        def exch(src, dst, nbr):
            i = sem_ctr[0]
            sem_ctr[0] += 1
            return pltpu.make_async_remote_copy(
                src_ref=src, dst_ref=dst,
                send_sem=send_sems.at[i], recv_sem=recv_sems.at[i],
                device_id=nbr, device_id_type=pl.DeviceIdType.MESH,
            )

        def col(c):
            return pl.ds(c * cw, cw)

        barrier = pltpu.get_barrier_semaphore()
        for nbr in (y_nbr, x_nbr):
            pl.semaphore_signal(
                barrier, inc=1, device_id=nbr,
                device_id_type=pl.DeviceIdType.MESH,
            )
        pl.semaphore_wait(barrier, 2)

        sa = (1 - my_y) * q
        oa = my_y * q
        sb = h + (1 - my_x) * q
        ob = h + my_x * q
        a_blk = oa + my_x * e
        b_blk = ob + my_y * e

        a1, b1 = {}, {}
        for p, arow in ((0, (1 - my_x) * e), (1, my_x * e)):
            for c in range(NC):
                a1[p, c] = exch(x_ref.at[pl.ds(sa + arow, e), col(c)],
                                rv_a1.at[pl.ds(arow, e), col(c)], y_nbr)
        for p, brow in ((0, (1 - my_y) * e), (1, my_y * e)):
            for c in range(NC):
                b1[p, c] = exch(x_ref.at[pl.ds(sb + brow, e), col(c)],
                                rv_b1.at[pl.ds(brow, e), col(c)], x_nbr)
        for c in range(NC):
            a1[0, c].start()
            b1[0, c].start()
        for c in range(NC):
            a1[1, c].start()
            b1[1, c].start()

        a2, b2 = {}, {}
        for c in range(NC):
            a1[0, c].wait_recv()
            acc_a[pl.ds((1 - my_x) * e, e), col(c)] = (
                x_ref[pl.ds(oa + (1 - my_x) * e, e), col(c)]
                + rv_a1[pl.ds((1 - my_x) * e, e), col(c)])
            a2[c] = exch(acc_a.at[pl.ds((1 - my_x) * e, e), col(c)],
                         rv_a2.at[:, col(c)], x_nbr)
            a2[c].start()

            b1[0, c].wait_recv()
            acc_b[pl.ds((1 - my_y) * e, e), col(c)] = (
                x_ref[pl.ds(ob + (1 - my_y) * e, e), col(c)]
                + rv_b1[pl.ds((1 - my_y) * e, e), col(c)])
            b2[c] = exch(acc_b.at[pl.ds((1 - my_y) * e, e), col(c)],
                         rv_b2.at[:, col(c)], y_nbr)
            b2[c].start()

        for c in range(NC):
            a1[1, c].wait_recv()
            acc_a[pl.ds(my_x * e, e), col(c)] = (
                x_ref[pl.ds(oa + my_x * e, e), col(c)]
                + rv_a1[pl.ds(my_x * e, e), col(c)])
            b1[1, c].wait_recv()
            acc_b[pl.ds(my_y * e, e), col(c)] = (
                x_ref[pl.ds(ob + my_y * e, e), col(c)]
                + rv_b1[pl.ds(my_y * e, e), col(c)])

        a3, b3, a4p1, b4p1 = {}, {}, {}, {}
        for c in range(NC):
            a2[c].wait_recv()
            out_ref[pl.ds(a_blk, e), col(c)] = (
                acc_a[pl.ds(my_x * e, e), col(c)] + rv_a2[:, col(c)])
            a4p1[c] = exch(out_ref.at[pl.ds(a_blk, e), col(c)],
                           out_ref.at[pl.ds(a_blk, e), col(c)], y_nbr)
            a4p1[c].start()
            a3[c] = exch(out_ref.at[pl.ds(a_blk, e), col(c)],
                         out_ref.at[pl.ds(a_blk, e), col(c)], x_nbr)
            a3[c].start()

            b2[c].wait_recv()
            out_ref[pl.ds(b_blk, e), col(c)] = (
                acc_b[pl.ds(my_y * e, e), col(c)] + rv_b2[:, col(c)])
            b4p1[c] = exch(out_ref.at[pl.ds(b_blk, e), col(c)],
                           out_ref.at[pl.ds(b_blk, e), col(c)], x_nbr)
            b4p1[c].start()
            b3[c] = exch(out_ref.at[pl.ds(b_blk, e), col(c)],
                         out_ref.at[pl.ds(b_blk, e), col(c)], y_nbr)
            b3[c].start()

        a4p2, b4p2 = {}, {}
        for c in range(NC):
            a3[c].wait_recv()
            a4p2[c] = exch(
                out_ref.at[pl.ds(oa + (1 - my_x) * e, e), col(c)],
                out_ref.at[pl.ds(oa + (1 - my_x) * e, e), col(c)], y_nbr)
            a4p2[c].start()
            b3[c].wait_recv()
            b4p2[c] = exch(
                out_ref.at[pl.ds(ob + (1 - my_y) * e, e), col(c)],
                out_ref.at[pl.ds(ob + (1 - my_y) * e, e), col(c)], x_nbr)
            b4p2[c].start()

        for c in range(NC):
            a4p1[c].wait_recv()
            b4p1[c].wait_recv()
            a4p2[c].wait_recv()
            b4p2[c].wait_recv()

        for r in (list(a1.values()) + list(b1.values())
                  + list(a2.values()) + list(b2.values())
                  + list(a3.values()) + list(b3.values())
                  + list(a4p1.values()) + list(b4p1.values())
                  + list(a4p2.values()) + list(b4p2.values())):
            r.wait_send()

    n_sems = 12 * NC
    return pl.pallas_call(
        body,
        out_shape=jax.ShapeDtypeStruct((m, n), jnp.float32),
        in_specs=[pl.BlockSpec(memory_space=pltpu.VMEM)],
        out_specs=pl.BlockSpec(memory_space=pltpu.VMEM),
        scratch_shapes=[
            pltpu.VMEM((q, n), jnp.float32),
            pltpu.VMEM((q, n), jnp.float32),
            pltpu.VMEM((q, n), jnp.float32),
            pltpu.VMEM((q, n), jnp.float32),
            pltpu.VMEM((e, n), jnp.float32),
            pltpu.VMEM((e, n), jnp.float32),
            pltpu.SemaphoreType.DMA((n_sems,)),
            pltpu.SemaphoreType.DMA((n_sems,)),
        ],
        compiler_params=pltpu.CompilerParams(collective_id=0),
    )(x)


# device time: 79405 ns/iter; 2.4475x vs baseline; 1.0002x over previous
import jax
import jax.numpy as jnp
from jax import lax
from jax.experimental import pallas as pl
from jax.experimental.pallas import tpu as pltpu

NC = 2


def kernel(x):
    m, n = x.shape
    h = m // 2
    q = m // 4
    e = m // 8
    ec = e // NC

    def body(x_ref, out_ref, fwd_a, fwd_b, rv_a1, rv_b1, rv_a2, rv_b2,
             send_sems, recv_sems):
        my_x = lax.axis_index("x")
        my_y = lax.axis_index("y")
        y_nbr = (my_x, 1 - my_y)
        x_nbr = (1 - my_x, my_y)

        sem_ctr = [0]

        def exch(src, dst, nbr):
            i = sem_ctr[0]
            sem_ctr[0] += 1
            return pltpu.make_async_remote_copy(
                src_ref=src, dst_ref=dst,
                send_sem=send_sems.at[i], recv_sem=recv_sems.at[i],
                device_id=nbr, device_id_type=pl.DeviceIdType.MESH,
            )

        barrier = pltpu.get_barrier_semaphore()
        for nbr in (y_nbr, x_nbr):
            pl.semaphore_signal(
                barrier, inc=1, device_id=nbr,
                device_id_type=pl.DeviceIdType.MESH,
            )
        pl.semaphore_wait(barrier, 2)

        sa = (1 - my_y) * q
        oa = my_y * q
        sb = h + (1 - my_x) * q
        ob = h + my_x * q
        fa = (1 - my_x) * e
        na = my_x * e
        fb = (1 - my_y) * e
        nb = my_y * e
        a_blk = oa + na
        b_blk = ob + nb

        def ds(base, c):
            return pl.ds(base + c * ec, ec)

        a1, b1 = {}, {}
        for p, arow in ((0, fa), (1, na)):
            for c in range(NC):
                a1[p, c] = exch(x_ref.at[ds(sa + arow, c)],
                                rv_a1.at[ds(arow, c)], y_nbr)
        for p, brow in ((0, fb), (1, nb)):
            for c in range(NC):
                b1[p, c] = exch(x_ref.at[ds(sb + brow, c)],
                                rv_b1.at[ds(brow, c)], x_nbr)
        for c in range(NC):
            a1[0, c].start()
            b1[0, c].start()
        for c in range(NC):
            a1[1, c].start()
            b1[1, c].start()

        a2, b2 = {}, {}
        for c in range(NC):
            a1[0, c].wait_recv()
            fwd_a[ds(0, c), :] = (
                x_ref[ds(oa + fa, c), :] + rv_a1[ds(fa, c), :])
            a2[c] = exch(fwd_a.at[ds(0, c)], rv_a2.at[ds(0, c)], x_nbr)
            a2[c].start()

            b1[0, c].wait_recv()
            fwd_b[ds(0, c), :] = (
                x_ref[ds(ob + fb, c), :] + rv_b1[ds(fb, c), :])
            b2[c] = exch(fwd_b.at[ds(0, c)], rv_b2.at[ds(0, c)], y_nbr)
            b2[c].start()

        a3, b3, a4p1, b4p1 = {}, {}, {}, {}
        for c in range(NC):
            a1[1, c].wait_recv()
            a2[c].wait_recv()
            out_ref[ds(a_blk, c), :] = (
                x_ref[ds(oa + na, c), :] + rv_a1[ds(na, c), :]
                + rv_a2[ds(0, c), :])
            a4p1[c] = exch(out_ref.at[ds(a_blk, c)],
                           out_ref.at[ds(a_blk, c)], y_nbr)
            a4p1[c].start()
            a3[c] = exch(out_ref.at[ds(a_blk, c)],
                         out_ref.at[ds(a_blk, c)], x_nbr)
            a3[c].start()

            b1[1, c].wait_recv()
            b2[c].wait_recv()
            out_ref[ds(b_blk, c), :] = (
                x_ref[ds(ob + nb, c), :] + rv_b1[ds(nb, c), :]
                + rv_b2[ds(0, c), :])
            b4p1[c] = exch(out_ref.at[ds(b_blk, c)],
                           out_ref.at[ds(b_blk, c)], x_nbr)
            b4p1[c].start()
            b3[c] = exch(out_ref.at[ds(b_blk, c)],
                         out_ref.at[ds(b_blk, c)], y_nbr)
            b3[c].start()

        a4p2, b4p2 = {}, {}
        for c in range(NC):
            a3[c].wait_recv()
            a4p2[c] = exch(out_ref.at[ds(oa + fa, c)],
                           out_ref.at[ds(oa + fa, c)], y_nbr)
            a4p2[c].start()
            b3[c].wait_recv()
            b4p2[c] = exch(out_ref.at[ds(ob + fb, c)],
                           out_ref.at[ds(ob + fb, c)], x_nbr)
            b4p2[c].start()

        for c in range(NC):
            a4p1[c].wait_recv()
            b4p1[c].wait_recv()
            a4p2[c].wait_recv()
            b4p2[c].wait_recv()

        for grp in (a1, b1, a2, b2, a3, b3, a4p1, b4p1, a4p2, b4p2):
            for r in grp.values():
                r.wait_send()

    n_sems = 12 * NC
    return pl.pallas_call(
        body,
        out_shape=jax.ShapeDtypeStruct((m, n), jnp.float32),
        in_specs=[pl.BlockSpec(memory_space=pltpu.VMEM)],
        out_specs=pl.BlockSpec(memory_space=pltpu.VMEM),
        scratch_shapes=[
            pltpu.VMEM((e, n), jnp.float32),
            pltpu.VMEM((e, n), jnp.float32),
            pltpu.VMEM((q, n), jnp.float32),
            pltpu.VMEM((q, n), jnp.float32),
            pltpu.VMEM((e, n), jnp.float32),
            pltpu.VMEM((e, n), jnp.float32),
            pltpu.SemaphoreType.DMA((n_sems,)),
            pltpu.SemaphoreType.DMA((n_sems,)),
        ],
        compiler_params=pltpu.CompilerParams(collective_id=0),
    )(x)
